# baseline (speedup 1.0000x reference)
import functools

import jax
import jax.numpy as jnp
from jax import lax
from jax.experimental import pallas as pl
from jax.experimental.pallas import tpu as pltpu
from jax.experimental.pallas import tpu_sc as plsc

N_ROWS = 320000
N_FEAT = 128
N_SEG = 10000
BLK = 128
RB = BLK
N_BLKS = N_ROWS // RB
N_TILES = 32
BASE_BLKS = N_BLKS // N_TILES
EXTRA = N_BLKS - BASE_BLKS * N_TILES
SLAB = 640
LAST_SLAB = N_SEG - 15 * SLAB

_mesh = plsc.VectorSubcoreMesh(core_axis_name="core", subcore_axis_name="subcore")


@functools.partial(
    pl.kernel,
    out_type=(
        jax.ShapeDtypeStruct((2, N_SEG, N_FEAT), jnp.float32),
        jax.ShapeDtypeStruct((N_SEG,), jnp.float32),
        jax.ShapeDtypeStruct((N_SEG,), jnp.float32),
    ),
    mesh=_mesh,
    scratch_types=[
        pltpu.VMEM_SHARED((N_SEG, N_FEAT), jnp.float32),
        pltpu.VMEM_SHARED((N_SEG,), jnp.float32),
        pltpu.VMEM((16, N_FEAT), jnp.float32),
        pltpu.VMEM((SLAB,), jnp.float32),
        pltpu.VMEM((BLK,), jnp.float32),
        pltpu.SemaphoreType.DMA,
        pltpu.SemaphoreType.DMA,
        pltpu.SemaphoreType.DMA,
    ],
)
def _sc_segment_sum(x_hbm, b_hbm, sums_hbm, cnts0_hbm, cnts1_hbm,
                    sums_sh, cnts_sh,
                    zbuf, zbuf1, ones_v, ld_sem0, ld_sem1, scat_sem):
    c = lax.axis_index("core")
    s = lax.axis_index("subcore")
    wid = s * 2 + c
    zero16 = jnp.zeros((16,), jnp.float32)
    one16 = jnp.ones((16,), jnp.float32)

    @pl.loop(0, 16)
    def _(r):
        for j in range(N_FEAT // 16):
            zbuf[r, pl.ds(j * 16, 16)] = zero16

    @pl.loop(0, SLAB // 16)
    def _(i):
        zbuf1[pl.ds(i * 16, 16)] = zero16

    for j in range(BLK // 16):
        ones_v[pl.ds(j * 16, 16)] = one16

    off = s * SLAB

    @pl.when(s < 15)
    def _():
        for k in range(SLAB // 16):
            pltpu.sync_copy(zbuf, sums_sh.at[pl.ds(off + k * 16, 16)])
        pltpu.sync_copy(zbuf1, cnts_sh.at[pl.ds(off, SLAB)])

    @pl.when(s == 15)
    def _():
        for k in range(LAST_SLAB // 16):
            pltpu.sync_copy(zbuf, sums_sh.at[pl.ds(off + k * 16, 16)])
        pltpu.sync_copy(zbuf1.at[pl.ds(0, LAST_SLAB)],
                        cnts_sh.at[pl.ds(off, LAST_SLAB)])

    plsc.subcore_barrier()

    base = wid * BASE_BLKS + jnp.minimum(wid, EXTRA)

    def main_phase(xb0, xb1, ib0, ib1):
        xbufs = (xb0, xb1)
        ibufs = (ib0, ib1)

        ld_sems = (ld_sem0, ld_sem1)

        def start_load(j, p):
            blk = base + j
            lx = pltpu.async_copy(x_hbm.at[pl.ds(blk * RB, RB)], xbufs[p],
                                  ld_sems[p])
            li = pltpu.async_copy(b_hbm.at[blk], ibufs[p], ld_sems[p])
            return (lx, li)

        def start_scat(p):
            return (
                pltpu.async_copy(xbufs[p], sums_sh.at[ibufs[p].at[0]],
                                 scat_sem, add=True),
                pltpu.async_copy(ones_v, cnts_sh.at[ibufs[p].at[0]], scat_sem,
                                 add=True),
            )

        pending_loads = start_load(0, 0)
        pending_scats = None
        for j in range(BASE_BLKS):
            p = j % 2
            if pending_scats is not None:
                for h in pending_scats:
                    h.wait()
            if j + 1 < BASE_BLKS:
                next_loads = start_load(j + 1, 1 - p)
            else:
                next_loads = None
            for h in pending_loads:
                h.wait()
            pending_scats = start_scat(p)
            pending_loads = next_loads
        for h in pending_scats:
            h.wait()

        @pl.when(wid < EXTRA)
        def _():
            blk = base + BASE_BLKS
            pltpu.sync_copy(x_hbm.at[pl.ds(blk * RB, RB)], xb0)
            pltpu.sync_copy(b_hbm.at[blk], ib0)
            pltpu.sync_copy(xb0, sums_sh.at[ib0.at[0]], add=True)
            pltpu.sync_copy(ones_v, cnts_sh.at[ib0.at[0]], add=True)

    pl.run_scoped(
        main_phase,
        pltpu.VMEM((RB, N_FEAT), jnp.float32),
        pltpu.VMEM((RB, N_FEAT), jnp.float32),
        pltpu.VMEM((1, BLK), jnp.int32),
        pltpu.VMEM((1, BLK), jnp.int32),
    )

    plsc.subcore_barrier()

    for core_id, cnts_hbm in ((0, cnts0_hbm), (1, cnts1_hbm)):
        @pl.when((c == core_id) & (s < 15))
        def _():
            pltpu.sync_copy(sums_sh.at[pl.ds(off, SLAB)],
                            sums_hbm.at[c, pl.ds(off, SLAB)])
            pltpu.sync_copy(cnts_sh.at[pl.ds(off, SLAB)], zbuf1)
            pltpu.sync_copy(zbuf1, cnts_hbm.at[pl.ds(off, SLAB)])

        @pl.when((c == core_id) & (s == 15))
        def _():
            pltpu.sync_copy(sums_sh.at[pl.ds(off, LAST_SLAB)],
                            sums_hbm.at[c, pl.ds(off, LAST_SLAB)])
            pltpu.sync_copy(cnts_sh.at[pl.ds(off, LAST_SLAB)],
                            zbuf1.at[pl.ds(0, LAST_SLAB)])
            pltpu.sync_copy(zbuf1.at[pl.ds(0, LAST_SLAB)],
                            cnts_hbm.at[pl.ds(off, LAST_SLAB)])


def _div_body(s_ref, c0_ref, c1_ref, o_ref):
    sm = s_ref[0] + s_ref[1]
    ct = jnp.maximum(c0_ref[...] + c1_ref[...], 1.0)
    o_ref[...] = sm / ct[:, None]


_tc_divide = pl.pallas_call(
    _div_body,
    out_shape=jax.ShapeDtypeStruct((N_SEG, N_FEAT), jnp.float32),
)


def kernel(x, batch):
    b32 = batch.astype(jnp.int32).reshape(N_BLKS, 1, BLK)
    sums, cnts0, cnts1 = _sc_segment_sum(x, b32)
    return _tc_divide(sums, cnts0, cnts1)


# speedup vs baseline: 1.0348x; 1.0348x over previous
import functools

import jax
import jax.numpy as jnp
from jax import lax
from jax.experimental import pallas as pl
from jax.experimental.pallas import tpu as pltpu
from jax.experimental.pallas import tpu_sc as plsc

N_ROWS = 320000
N_FEAT = 128
N_SEG = 10000
BLK = 128
RB = BLK
N_BLKS = N_ROWS // RB
N_TILES = 32
BASE_BLKS = N_BLKS // N_TILES
EXTRA = N_BLKS - BASE_BLKS * N_TILES
SLAB = 640
LAST_SLAB = N_SEG - 15 * SLAB

_mesh = plsc.VectorSubcoreMesh(core_axis_name="core", subcore_axis_name="subcore")


@functools.partial(
    pl.kernel,
    out_type=(
        jax.ShapeDtypeStruct((2, N_SEG, N_FEAT), jnp.float32),
        jax.ShapeDtypeStruct((N_SEG,), jnp.float32),
        jax.ShapeDtypeStruct((N_SEG,), jnp.float32),
    ),
    mesh=_mesh,
    scratch_types=[
        pltpu.VMEM_SHARED((N_SEG, N_FEAT), jnp.float32),
        pltpu.VMEM_SHARED((N_SEG,), jnp.float32),
        pltpu.VMEM((SLAB,), jnp.float32),
        pltpu.VMEM((BLK,), jnp.float32),
        pltpu.SemaphoreType.DMA,
        pltpu.SemaphoreType.DMA,
        pltpu.SemaphoreType.DMA,
        pltpu.SemaphoreType.DMA,
        pltpu.SemaphoreType.DMA,
        pltpu.SemaphoreType.DMA,
    ],
)
def _sc_segment_sum(x_hbm, b_hbm, sums_hbm, cnts0_hbm, cnts1_hbm,
                    sums_sh, cnts_sh, zbuf1, ones_v,
                    ls0, ls1, ls2, ss0, ss1, ss2):
    c = lax.axis_index("core")
    s = lax.axis_index("subcore")
    wid = s * 2 + c
    zero16 = jnp.zeros((16,), jnp.float32)
    one16 = jnp.ones((16,), jnp.float32)

    @pl.loop(0, SLAB // 16)
    def _(i):
        zbuf1[pl.ds(i * 16, 16)] = zero16

    for j in range(BLK // 16):
        ones_v[pl.ds(j * 16, 16)] = one16

    off = s * SLAB
    base = wid * BASE_BLKS + jnp.minimum(wid, EXTRA)
    N_CHUNKS = BASE_BLKS // 3

    def main_phase(xb0, xb1, xb2, ib0, ib1, ib2):
        xbufs = (xb0, xb1, xb2)
        ibufs = (ib0, ib1, ib2)
        lsems = (ls0, ls1, ls2)
        ssems = (ss0, ss1, ss2)

        @pl.loop(0, RB)
        def _(r):
            for j in range(N_FEAT // 16):
                xb0[r, pl.ds(j * 16, 16)] = zero16

        @pl.when(s < 15)
        def _():
            for k in range(SLAB // RB):
                pltpu.sync_copy(xb0, sums_sh.at[pl.ds(off + k * RB, RB)])
            pltpu.sync_copy(zbuf1, cnts_sh.at[pl.ds(off, SLAB)])

        @pl.when(s == 15)
        def _():
            for k in range(LAST_SLAB // RB):
                pltpu.sync_copy(xb0, sums_sh.at[pl.ds(off + k * RB, RB)])
            rem = LAST_SLAB % RB
            if rem:
                pltpu.sync_copy(xb0.at[pl.ds(0, rem)],
                                sums_sh.at[pl.ds(off + LAST_SLAB - rem, rem)])
            pltpu.sync_copy(zbuf1.at[pl.ds(0, LAST_SLAB)],
                            cnts_sh.at[pl.ds(off, LAST_SLAB)])

        plsc.subcore_barrier()

        def start_load(blk, q):
            pltpu.async_copy(x_hbm.at[pl.ds(blk * RB, RB)], xbufs[q],
                             lsems[q])
            pltpu.async_copy(b_hbm.at[blk], ibufs[q], lsems[q])

        def drain_load(q):
            pltpu.make_async_copy(x_hbm.at[pl.ds(0, RB)], xbufs[q],
                                  lsems[q]).wait()
            pltpu.make_async_copy(b_hbm.at[0], ibufs[q], lsems[q]).wait()

        def start_scat(q):
            pltpu.async_copy(xbufs[q], sums_sh.at[ibufs[q].at[0]], ssems[q],
                             add=True)
            pltpu.async_copy(ones_v, cnts_sh.at[ibufs[q].at[0]], ssems[q],
                             add=True)

        def drain_scat(q):
            pltpu.make_async_copy(xbufs[q], sums_sh.at[ibufs[q].at[0]],
                                  ssems[q]).wait()
            pltpu.make_async_copy(ones_v, cnts_sh.at[ibufs[q].at[0]],
                                  ssems[q]).wait()

        start_load(base, 0)
        start_load(base + 1, 1)

        @pl.loop(0, N_CHUNKS)
        def _(k):
            j0 = base + 3 * k
            drain_load(0)
            start_scat(0)

            @pl.when(k > 0)
            def _():
                drain_scat(2)

            start_load(j0 + 2, 2)
            drain_load(1)
            start_scat(1)
            drain_scat(0)

            @pl.when(k < N_CHUNKS - 1)
            def _():
                start_load(j0 + 3, 0)

            drain_load(2)
            start_scat(2)
            drain_scat(1)

            @pl.when(k < N_CHUNKS - 1)
            def _():
                start_load(j0 + 4, 1)

        drain_scat(2)

        @pl.when(wid < EXTRA)
        def _():
            blk = base + BASE_BLKS
            pltpu.sync_copy(x_hbm.at[pl.ds(blk * RB, RB)], xb0)
            pltpu.sync_copy(b_hbm.at[blk], ib0)
            pltpu.sync_copy(xb0, sums_sh.at[ib0.at[0]], add=True)
            pltpu.sync_copy(ones_v, cnts_sh.at[ib0.at[0]], add=True)

    pl.run_scoped(
        main_phase,
        pltpu.VMEM((RB, N_FEAT), jnp.float32),
        pltpu.VMEM((RB, N_FEAT), jnp.float32),
        pltpu.VMEM((RB, N_FEAT), jnp.float32),
        pltpu.VMEM((1, BLK), jnp.int32),
        pltpu.VMEM((1, BLK), jnp.int32),
        pltpu.VMEM((1, BLK), jnp.int32),
    )

    plsc.subcore_barrier()

    for core_id, cnts_hbm in ((0, cnts0_hbm), (1, cnts1_hbm)):
        @pl.when((c == core_id) & (s < 15))
        def _():
            pltpu.sync_copy(sums_sh.at[pl.ds(off, SLAB)],
                            sums_hbm.at[c, pl.ds(off, SLAB)])
            pltpu.sync_copy(cnts_sh.at[pl.ds(off, SLAB)], zbuf1)
            pltpu.sync_copy(zbuf1, cnts_hbm.at[pl.ds(off, SLAB)])

        @pl.when((c == core_id) & (s == 15))
        def _():
            pltpu.sync_copy(sums_sh.at[pl.ds(off, LAST_SLAB)],
                            sums_hbm.at[c, pl.ds(off, LAST_SLAB)])
            pltpu.sync_copy(cnts_sh.at[pl.ds(off, LAST_SLAB)],
                            zbuf1.at[pl.ds(0, LAST_SLAB)])
            pltpu.sync_copy(zbuf1.at[pl.ds(0, LAST_SLAB)],
                            cnts_hbm.at[pl.ds(off, LAST_SLAB)])


def _div_body(s_ref, c0_ref, c1_ref, o_ref):
    sm = s_ref[0] + s_ref[1]
    ct = jnp.maximum(c0_ref[...] + c1_ref[...], 1.0)
    o_ref[...] = sm / ct[:, None]


_tc_divide = pl.pallas_call(
    _div_body,
    out_shape=jax.ShapeDtypeStruct((N_SEG, N_FEAT), jnp.float32),
)


def kernel(x, batch):
    b32 = batch.astype(jnp.int32).reshape(N_BLKS, 1, BLK)
    sums, cnts0, cnts1 = _sc_segment_sum(x, b32)
    return _tc_divide(sums, cnts0, cnts1)


# speedup vs baseline: 1.0570x; 1.0215x over previous
import functools

import jax
import jax.numpy as jnp
from jax import lax
from jax.experimental import pallas as pl
from jax.experimental.pallas import tpu as pltpu
from jax.experimental.pallas import tpu_sc as plsc

N_ROWS = 320000
N_FEAT = 128
N_SEG = 10000
BLK = 128
N_BLKS = N_ROWS // BLK
SLAB = 640
LAST_SLAB = N_SEG - 15 * SLAB

_mesh = plsc.VectorSubcoreMesh(core_axis_name="core", subcore_axis_name="subcore")


@functools.partial(
    pl.kernel,
    out_type=(
        jax.ShapeDtypeStruct((2, N_SEG, N_FEAT), jnp.float32),
        jax.ShapeDtypeStruct((N_SEG,), jnp.float32),
        jax.ShapeDtypeStruct((N_SEG,), jnp.float32),
    ),
    mesh=_mesh,
    scratch_types=[
        pltpu.VMEM_SHARED((N_SEG, N_FEAT), jnp.float32),
        pltpu.VMEM_SHARED((N_SEG,), jnp.float32),
        pltpu.VMEM((64, N_FEAT), jnp.float32),
        pltpu.VMEM((SLAB,), jnp.float32),
        pltpu.VMEM((BLK,), jnp.float32),
        pltpu.SemaphoreType.DMA,
    ],
)
def _sc_segment_sum(x_hbm, b_hbm, sums_hbm, cnts0_hbm, cnts1_hbm,
                    sums_sh, cnts_sh, zbuf, zbuf1, ones_v, scat_sem):
    c = lax.axis_index("core")
    s = lax.axis_index("subcore")
    zero16 = jnp.zeros((16,), jnp.float32)
    one16 = jnp.ones((16,), jnp.float32)

    @pl.loop(0, 64)
    def _(r):
        for j in range(8):
            zbuf[r, pl.ds(j * 16, 16)] = zero16

    @pl.loop(0, SLAB // 16)
    def _(i):
        zbuf1[pl.ds(i * 16, 16)] = zero16

    for j in range(BLK // 16):
        ones_v[pl.ds(j * 16, 16)] = one16

    off = s * SLAB

    @pl.when(s < 15)
    def _():
        for k in range(SLAB // 64):
            pltpu.sync_copy(zbuf, sums_sh.at[pl.ds(off + k * 64, 64)])
        pltpu.sync_copy(zbuf1, cnts_sh.at[pl.ds(off, SLAB)])

    @pl.when(s == 15)
    def _():
        for k in range(LAST_SLAB // 64):
            pltpu.sync_copy(zbuf, sums_sh.at[pl.ds(off + k * 64, 64)])
        rem = LAST_SLAB % 64
        if rem:
            pltpu.sync_copy(zbuf.at[pl.ds(0, rem)],
                            sums_sh.at[pl.ds(off + LAST_SLAB - rem, rem)])
        pltpu.sync_copy(zbuf1.at[pl.ds(0, LAST_SLAB)],
                        cnts_sh.at[pl.ds(off, LAST_SLAB)])

    plsc.subcore_barrier()

    def scat_body(x_v, i_v):
        a = pltpu.async_copy(x_v, sums_sh.at[i_v.at[0]], scat_sem, add=True)
        b = pltpu.async_copy(ones_v, cnts_sh.at[i_v.at[0]], scat_sem, add=True)
        a.wait()
        b.wait()

    pltpu.emit_pipeline(
        scat_body,
        grid=(N_BLKS,),
        in_specs=[
            pl.BlockSpec((BLK, N_FEAT), lambda i: (i, 0)),
            pl.BlockSpec((1, BLK), lambda i: (i, 0)),
        ],
        core_axis_name=("core", "subcore"),
        dimension_semantics=(pltpu.PARALLEL,),
        trace_scopes=False,
    )(x_hbm, b_hbm)

    plsc.subcore_barrier()

    for core_id, cnts_hbm in ((0, cnts0_hbm), (1, cnts1_hbm)):
        @pl.when((c == core_id) & (s < 15))
        def _():
            pltpu.sync_copy(sums_sh.at[pl.ds(off, SLAB)],
                            sums_hbm.at[c, pl.ds(off, SLAB)])
            pltpu.sync_copy(cnts_sh.at[pl.ds(off, SLAB)], zbuf1)
            pltpu.sync_copy(zbuf1, cnts_hbm.at[pl.ds(off, SLAB)])

        @pl.when((c == core_id) & (s == 15))
        def _():
            pltpu.sync_copy(sums_sh.at[pl.ds(off, LAST_SLAB)],
                            sums_hbm.at[c, pl.ds(off, LAST_SLAB)])
            pltpu.sync_copy(cnts_sh.at[pl.ds(off, LAST_SLAB)],
                            zbuf1.at[pl.ds(0, LAST_SLAB)])
            pltpu.sync_copy(zbuf1.at[pl.ds(0, LAST_SLAB)],
                            cnts_hbm.at[pl.ds(off, LAST_SLAB)])


def _div_body(s_ref, c0_ref, c1_ref, o_ref):
    sm = s_ref[0] + s_ref[1]
    ct = jnp.maximum(c0_ref[...] + c1_ref[...], 1.0)
    o_ref[...] = sm / ct[:, None]


_tc_divide = pl.pallas_call(
    _div_body,
    out_shape=jax.ShapeDtypeStruct((N_SEG, N_FEAT), jnp.float32),
)


def kernel(x, batch):
    b32 = batch.astype(jnp.int32).reshape(N_BLKS, BLK)
    sums, cnts0, cnts1 = _sc_segment_sum(x, b32)
    return _tc_divide(sums, cnts0, cnts1)
